# baseline (speedup 1.0000x reference)
import functools

import jax
import jax.numpy as jnp
from jax import lax
from jax.experimental import pallas as pl
from jax.experimental.pallas import tpu as pltpu
from jax.experimental.pallas import tpu_sc as plsc

B = 16384
D = 32
NC = 2
NS = 16
NW = NC * NS
BPW = B // NW
CHUNK = 128
NCH = BPW // CHUNK

_mesh = plsc.VectorSubcoreMesh(core_axis_name="c", subcore_axis_name="s")


@functools.partial(
    pl.kernel,
    mesh=_mesh,
    out_type=jax.ShapeDtypeStruct((B,), jnp.float32),
    compiler_params=pltpu.CompilerParams(
        needs_layout_passes=False, use_tc_tiling_on_sc=False),
    scratch_types=[
        pltpu.VMEM((NCH, CHUNK), jnp.int32),
        pltpu.VMEM((NCH, CHUNK), jnp.int32),
        pltpu.VMEM((BPW, D), jnp.float32),
        pltpu.VMEM((BPW, D), jnp.float32),
        pltpu.VMEM((BPW,), jnp.float32),
        pltpu.VMEM((BPW,), jnp.float32),
        pltpu.VMEM((BPW,), jnp.float32),
        pltpu.SemaphoreType.DMA,
    ],
)
def _mf_sc(users_hbm, items_hbm, ue_hbm, ie_hbm, ub_hbm, ib_hbm, out_hbm,
           idx_u, idx_i, u_rows, v_rows, bu_v, bi_v, out_v, sem):
    wid = lax.axis_index("s") * NC + lax.axis_index("c")

    pltpu.sync_copy(users_hbm.at[pl.ds(wid * NCH, NCH)], idx_u)
    pltpu.sync_copy(items_hbm.at[pl.ds(wid * NCH, NCH)], idx_i)

    copies = []
    for j in range(NCH):
        dst = pl.ds(j * CHUNK, CHUNK)
        copies.append(pltpu.async_copy(ue_hbm.at[idx_u.at[j]], u_rows.at[dst], sem))
        copies.append(pltpu.async_copy(ie_hbm.at[idx_i.at[j]], v_rows.at[dst], sem))
        copies.append(pltpu.async_copy(ub_hbm.at[idx_u.at[j]], bu_v.at[dst], sem))
        copies.append(pltpu.async_copy(ib_hbm.at[idx_i.at[j]], bi_v.at[dst], sem))
    for c in copies:
        c.wait()

    def body(i, carry):
        r0 = i * 16
        rows = r0 + lax.iota(jnp.int32, 16)
        acc = bu_v[pl.ds(r0, 16)] + bi_v[pl.ds(r0, 16)]
        for d in range(D):
            dd = jnp.full((16,), d, jnp.int32)
            ucol = plsc.load_gather(u_rows, [rows, dd])
            vcol = plsc.load_gather(v_rows, [rows, dd])
            acc = acc + ucol * vcol
        out_v[pl.ds(r0, 16)] = acc
        return carry

    lax.fori_loop(0, BPW // 16, body, 0)

    pltpu.sync_copy(out_v, out_hbm.at[pl.ds(wid * BPW, BPW)])


def kernel(users, items, user_emb, item_emb, user_bias, item_bias):
    users2 = users.astype(jnp.int32).reshape(B // CHUNK, CHUNK)
    items2 = items.astype(jnp.int32).reshape(B // CHUNK, CHUNK)
    return _mf_sc(users2, items2, user_emb, item_emb,
                  user_bias.reshape(-1), item_bias.reshape(-1))
